# Initial kernel scaffold; baseline (speedup 1.0000x reference)
import jax
import jax.numpy as jnp
from jax.experimental import pallas as pl


def kernel(output, target):
    raise NotImplementedError("write your pallas kernel here")


# speedup vs baseline: 5.8316x; 5.8316x over previous
import jax
import jax.numpy as jnp
from jax import lax
from jax.experimental import pallas as pl
from jax.experimental.pallas import tpu as pltpu

_ANCH = ((116.0, 90.0, 156.0, 198.0, 373.0, 326.0),
         (30.0, 61.0, 62.0, 45.0, 59.0, 119.0),
         (10.0, 13.0, 16.0, 30.0, 33.0, 23.0))
_RESO = 416.0
_NWS = (13, 26, 52)
_BASES = (0, 507, 2535)
_NANS = (507, 2028, 8112)
_A = 10647
_NT = 50
_NB = 16
_TH = 0.6


def _iou(x1, y1, w1, h1, x2, y2, w2, h2):
    ax1 = x1 - w1 / 2.0
    ax2 = x1 + w1 / 2.0
    ay1 = y1 - h1 / 2.0
    ay2 = y1 + h1 / 2.0
    bx1 = x2 - w2 / 2.0
    bx2 = x2 + w2 / 2.0
    by1 = y2 - h2 / 2.0
    by2 = y2 + h2 / 2.0
    iw = jnp.maximum(jnp.minimum(ax2, bx2) - jnp.maximum(ax1, bx1), 0.0)
    ih = jnp.maximum(jnp.minimum(ay2, by2) - jnp.maximum(ay1, by1), 0.0)
    inter = iw * ih
    union = (ax2 - ax1) * (ay2 - ay1) + (bx2 - bx1) * (by2 - by1) - inter
    return inter / jnp.maximum(union, 1e-16)


def _body(out_ref, trc_ref, trt_ref, loss_ref):
    blk4 = out_ref[0, :, 0:4] / _RESO
    px = blk4[:, 0:1]
    py = blk4[:, 1:2]
    pw = blk4[:, 2:3]
    ph = blk4[:, 3:4]
    base = 0.5 * (jnp.sum((px - 0.5) ** 2) + jnp.sum((py - 0.5) ** 2)
                  + jnp.sum(pw * pw) + jnp.sum(ph * ph))
    cls = out_ref[0, :, 5:85]
    base = base - jnp.sum(jnp.maximum(jnp.log(1.0 - cls), -100.0))
    conf = out_ref[0, :, 5:6]

    t0r = trt_ref[0, 0:1, :]
    t1r = trt_ref[0, 1:2, :]
    t2r = trt_ref[0, 2:3, :]
    t3r = trt_ref[0, 3:4, :]
    t4r = trt_ref[0, 4:5, :]
    trc = trc_ref[0]
    t0c = trc[:, 0:1]
    t1c = trc[:, 1:2]
    t2c = trc[:, 2:3]
    t3c = trc[:, 3:4]
    t4c = trc[:, 4:5]

    ii = lax.broadcasted_iota(jnp.int32, (_NT, _NT), 0)
    jj = lax.broadcasted_iota(jnp.int32, (_NT, _NT), 1)
    zr = (t1r == 0.0).astype(jnp.float32)
    zc = (t1c == 0.0).astype(jnp.float32)
    prefT = jnp.sum(jnp.where(ii <= jj, zc, 0.0), axis=0, keepdims=True)
    aliveT = prefT == 0.0
    prefC = jnp.sum(jnp.where(jj <= ii, zr, 0.0), axis=1, keepdims=True)
    aliveC = prefC == 0.0

    aidx = lax.broadcasted_iota(jnp.int32, (_A, 1), 0)
    nwv = jnp.where(aidx < _BASES[1], float(_NWS[0]),
                    jnp.where(aidx < _BASES[2], float(_NWS[1]), float(_NWS[2])))
    iou_all = _iou(px, py, pw, ph, t1r * nwv, t2r * nwv, t3r * nwv, t4r * nwv)
    iou_all = jnp.where(aliveT, iou_all, 0.0)
    cur = jnp.max(iou_all, axis=1, keepdims=True)
    base = base + jnp.sum(jnp.where(cur > _TH, 0.0, 12.5 * conf * conf))

    blk16 = out_ref[0, :, 0:16]
    iotaA = lax.broadcasted_iota(jnp.int32, (_NT, _A), 1)
    iota16 = lax.broadcasted_iota(jnp.int32, (_NT, 16), 1)
    corr = jnp.float32(0.0)
    for lvl in range(3):
        nw = float(_NWS[lvl])
        nwi = _NWS[lvl]
        nan = _NANS[lvl]
        off = _BASES[lvl]
        aw = _ANCH[lvl]

        def best_slot(t1, t2, t3, t4):
            gw = t3 * nw
            gh = t4 * nw
            i0 = _iou(0.0, 0.0, aw[0], aw[1], 0.0, 0.0, gw, gh)
            i1 = _iou(0.0, 0.0, aw[2], aw[3], 0.0, 0.0, gw, gh)
            i2 = _iou(0.0, 0.0, aw[4], aw[5], 0.0, 0.0, gw, gh)
            bn = jnp.where((i0 >= i1) & (i0 >= i2), 0, jnp.where(i1 >= i2, 1, 2))
            bmax = jnp.maximum(jnp.maximum(i0, i1), i2)
            bn = jnp.where(bmax > 0.0, bn, -1)
            gi = (t1 * nw).astype(jnp.int32)
            gj = (t2 * nw).astype(jnp.int32)
            idx = 3 * (nwi * gi + gj) + bn
            return jnp.where(idx < 0, idx + nan, idx)

        idxr = best_slot(t1r, t2r, t3r, t4r)
        idxc = best_slot(t1c, t2c, t3c, t4c)

        eq = idxc == idxr
        later = jnp.sum(jnp.where((jj > ii) & eq & aliveT, 1.0, 0.0),
                        axis=1, keepdims=True)
        winc = aliveC & (later == 0.0)
        keyc = idxc * 128 + t0c.astype(jnp.int32)
        keyr = idxr * 128 + t0r.astype(jnp.int32)
        eqk = keyc == keyr
        earlier = jnp.sum(jnp.where((jj < ii) & eqk & aliveT, 1.0, 0.0),
                          axis=1, keepdims=True)
        firstc = aliveC & (earlier == 0.0)

        oh = ((off + idxc) == iotaA).astype(jnp.float32)
        g = lax.dot_general(oh, blk16, (((1,), (0,)), ((), ())),
                            preferred_element_type=jnp.float32,
                            precision=lax.Precision.HIGHEST)
        pxs = g[:, 0:1] / _RESO
        pys = g[:, 1:2] / _RESO
        pws = g[:, 2:3] / _RESO
        phs = g[:, 3:4] / _RESO
        confs = g[:, 5:6]
        ccol = t0c.astype(jnp.int32)
        vcls = jnp.sum(jnp.where(iota16 == 5 + ccol, g, 0.0),
                       axis=1, keepdims=True)

        dcoord = 0.5 * ((pxs - t1c) ** 2 - (pxs - 0.5) ** 2
                        + (pys - t2c) ** 2 - (pys - 0.5) ** 2
                        + (pws - t3c) ** 2 - pws * pws
                        + (phs - t4c) ** 2 - phs * phs)
        tconf = _iou(t1c * nw, t2c * nw, t3c * nw, t4c * nw, pxs, pys, pws, phs)
        slot_all = _iou(pxs, pys, pws, phs, t1r * nw, t2r * nw, t3r * nw, t4r * nw)
        slot_all = jnp.where(aliveT, slot_all, 0.0)
        curs = jnp.max(slot_all, axis=1, keepdims=True)
        dconf = 0.5 * (confs - tconf) ** 2 - jnp.where(
            curs > _TH, 0.0, 12.5 * confs * confs)
        dcls = (-jnp.maximum(jnp.log(vcls), -100.0)
                + jnp.maximum(jnp.log(1.0 - vcls), -100.0))
        corr = corr + jnp.sum(jnp.where(winc, dcoord + dconf, 0.0))
        corr = corr + jnp.sum(jnp.where(firstc, dcls, 0.0))

    loss_ref[0, 0, 0] = base + corr


def kernel(output, target):
    trc = target.reshape(_NB, _NT, 5)
    trt = jnp.transpose(trc, (0, 2, 1))
    partial = pl.pallas_call(
        _body,
        grid=(_NB,),
        in_specs=[
            pl.BlockSpec((1, _A, 85), lambda b: (b, 0, 0)),
            pl.BlockSpec((1, _NT, 5), lambda b: (b, 0, 0)),
            pl.BlockSpec((1, 5, _NT), lambda b: (b, 0, 0)),
        ],
        out_specs=pl.BlockSpec((1, 1, 1), lambda b: (b, 0, 0),
                               memory_space=pltpu.SMEM),
        out_shape=jax.ShapeDtypeStruct((_NB, 1, 1), jnp.float32),
    )(output, trc, trt)
    return jnp.sum(partial)
